# Initial kernel scaffold; baseline (speedup 1.0000x reference)
import jax
import jax.numpy as jnp
from jax.experimental import pallas as pl


def kernel(batch_cls_preds, batch_box_preds):
    raise NotImplementedError("write your pallas kernel here")


# speedup vs baseline: 2.8514x; 2.8514x over previous
import functools

import jax
import jax.numpy as jnp
from jax import lax
from jax.experimental import pallas as pl
from jax.experimental.pallas import tpu as pltpu

SCORE_THRESH = 0.1
IOU_THRESH = 0.5
PRE_MAX = 1024
POST_MAX = 256
ROW_CHUNK = 128


def _score_body(cls_ref, masked_ref, labels_ref):
    c = cls_ref[0]
    c0, c1, c2 = c[0:1], c[1:2], c[2:3]
    s = jnp.maximum(jnp.maximum(c0, c1), c2)
    lab = jnp.where(
        c0 >= c1,
        jnp.where(c0 >= c2, 1.0, 3.0),
        jnp.where(c1 >= c2, 2.0, 3.0),
    )
    masked_ref[0] = jnp.where(s > SCORE_THRESH, s, -1.0)
    labels_ref[0] = lab


def _col_to_row(v_col, n=PRE_MAX, chunk=ROW_CHUNK):
    lane = lax.broadcasted_iota(jnp.int32, (chunk, n), 1)
    out = jnp.zeros((1, n), jnp.float32)
    for c in range(n // chunk):
        rid = lax.broadcasted_iota(jnp.int32, (chunk, n), 0) + c * chunk
        out = out + jnp.sum(
            jnp.where(rid == lane, v_col[c * chunk:(c + 1) * chunk], 0.0),
            axis=0, keepdims=True)
    return out


def _nms_body(tab_ref, idx_ref, ts_ref, boxes_ref, fs_ref, fl_ref,
              rows_ref, s_ref):
    def gbody(j, _):
        idx = idx_ref[0, 0, j]
        rows_ref[pl.ds(j, 1), :] = tab_ref[0, pl.ds(idx, 1), :]
        return 0

    lax.fori_loop(0, PRE_MAX, gbody, 0, unroll=8)

    x = rows_ref[:, 0:1]
    y = rows_ref[:, 1:2]
    dx = rows_ref[:, 3:4]
    dy = rows_ref[:, 4:5]
    x1c, x2c = x - dx * 0.5, x + dx * 0.5
    y1c, y2c = y - dy * 0.5, y + dy * 0.5
    areac = dx * dy
    x1r = _col_to_row(x1c)
    x2r = _col_to_row(x2c)
    y1r = _col_to_row(y1c)
    y2r = _col_to_row(y2c)
    arear = _col_to_row(areac)

    lane = lax.broadcasted_iota(jnp.int32, (ROW_CHUNK, PRE_MAX), 1)
    for c in range(PRE_MAX // ROW_CHUNK):
        sl = slice(c * ROW_CHUNK, (c + 1) * ROW_CHUNK)
        rid = lax.broadcasted_iota(jnp.int32, (ROW_CHUNK, PRE_MAX), 0) + c * ROW_CHUNK
        ix1 = jnp.maximum(x1c[sl], x1r)
        ix2 = jnp.minimum(x2c[sl], x2r)
        iy1 = jnp.maximum(y1c[sl], y1r)
        iy2 = jnp.minimum(y2c[sl], y2r)
        inter = jnp.maximum(ix2 - ix1, 0.0) * jnp.maximum(iy2 - iy1, 0.0)
        union = areac[sl] + arear - inter
        iou = inter / jnp.maximum(union, 1e-6)
        s_ref[sl, :] = jnp.where((iou > IOU_THRESH) & (lane > rid), 1.0, 0.0)

    lane1 = lax.broadcasted_iota(jnp.int32, (1, PRE_MAX), 1)

    def body(i, keepf):
        row = s_ref[pl.ds(i, 1), :]
        curf = jnp.max(jnp.where(lane1 == i, keepf, 0.0))
        return keepf * (1.0 - row * curf)

    keepf = lax.fori_loop(0, PRE_MAX, body, jnp.ones((1, PRE_MAX), jnp.float32))

    ts = ts_ref[0]
    sel = jnp.where((keepf > 0.5) & (ts > 0.0), ts, -1.0)
    sel_col = jnp.zeros((PRE_MAX, 1), jnp.float32)
    lane_c = lane
    rank = jnp.zeros((1, PRE_MAX), jnp.float32)
    for c in range(PRE_MAX // ROW_CHUNK):
        rid = lax.broadcasted_iota(jnp.int32, (ROW_CHUNK, PRE_MAX), 0) + c * ROW_CHUNK
        ident = rid == lane_c
        part = jnp.sum(jnp.where(ident, sel, 0.0), axis=1, keepdims=True)
        g = (part > sel) | ((part == sel) & (rid < lane_c))
        rank = rank + jnp.sum(g.astype(jnp.float32), axis=0, keepdims=True)

    rows_all = rows_ref[...]
    for c in range(POST_MAX // ROW_CHUNK):
        sl = slice(c * ROW_CHUNK, (c + 1) * ROW_CHUNK)
        kio = (lax.broadcasted_iota(jnp.int32, (ROW_CHUNK, PRE_MAX), 0)
               + c * ROW_CHUNK).astype(jnp.float32)
        m = rank == kio
        mf = m.astype(jnp.float32)
        fr = lax.dot_general(mf, rows_all, (((1,), (0,)), ((), ())),
                             preferred_element_type=jnp.float32)
        fsc = jnp.sum(jnp.where(m, sel, 0.0), axis=1, keepdims=True)
        boxes_ref[0, sl, :] = fr[:, 0:7]
        fs_ref[0, sl, :] = jnp.where(fsc > 0.0, fsc, 0.0)
        fl_ref[0, sl, :] = jnp.where(fsc > 0.0, fr[:, 7:8], 0.0).astype(jnp.int32)


@jax.jit
def kernel(batch_cls_preds, batch_box_preds):
    B, N, C = batch_cls_preds.shape
    cls_t = jnp.transpose(batch_cls_preds, (0, 2, 1))

    masked, labels_f = pl.pallas_call(
        _score_body,
        grid=(B,),
        in_specs=[pl.BlockSpec((1, C, N), lambda b: (b, 0, 0))],
        out_specs=[pl.BlockSpec((1, 1, N), lambda b: (b, 0, 0)),
                   pl.BlockSpec((1, 1, N), lambda b: (b, 0, 0))],
        out_shape=[jax.ShapeDtypeStruct((B, 1, N), jnp.float32),
                   jax.ShapeDtypeStruct((B, 1, N), jnp.float32)],
    )(cls_t)
    masked = masked.reshape(B, N)
    labels_f = labels_f.reshape(B, N)

    top_scores, top_idx = lax.top_k(masked, PRE_MAX)

    table = jnp.concatenate([batch_box_preds, labels_f[..., None]], axis=-1)

    fb, fs, fl = pl.pallas_call(
        _nms_body,
        grid=(B,),
        in_specs=[
            pl.BlockSpec((1, N, 8), lambda b: (b, 0, 0)),
            pl.BlockSpec((1, 1, PRE_MAX), lambda b: (b, 0, 0),
                         memory_space=pltpu.SMEM),
            pl.BlockSpec((1, 1, PRE_MAX), lambda b: (b, 0, 0)),
        ],
        out_specs=[
            pl.BlockSpec((1, POST_MAX, 7), lambda b: (b, 0, 0)),
            pl.BlockSpec((1, POST_MAX, 1), lambda b: (b, 0, 0)),
            pl.BlockSpec((1, POST_MAX, 1), lambda b: (b, 0, 0)),
        ],
        out_shape=[
            jax.ShapeDtypeStruct((B, POST_MAX, 7), jnp.float32),
            jax.ShapeDtypeStruct((B, POST_MAX, 1), jnp.float32),
            jax.ShapeDtypeStruct((B, POST_MAX, 1), jnp.int32),
        ],
        scratch_shapes=[
            pltpu.VMEM((PRE_MAX, 8), jnp.float32),
            pltpu.VMEM((PRE_MAX, PRE_MAX), jnp.float32),
        ],
    )(table, top_idx.reshape(B, 1, PRE_MAX),
      top_scores.reshape(B, 1, PRE_MAX))

    return fb, fs.reshape(B, POST_MAX), fl.reshape(B, POST_MAX)
